# initial kernel scaffold (unmeasured)
import jax
import jax.numpy as jnp
from jax import lax
from jax.experimental import pallas as pl
from jax.experimental.pallas import tpu as pltpu

N_DEV = 4


def _partial_matmul(x, w, scale_x, scale_w):
    M, K = x.shape
    _, N = w.shape
    BM, BN = 512, 2048

    def body(sx_ref, sw_ref, x_ref, w_ref, o_ref):
        a = x_ref[...].astype(jnp.bfloat16)
        b = w_ref[...].astype(jnp.bfloat16)
        acc = jax.lax.dot_general(
            a, b, (((1,), (0,)), ((), ())),
            preferred_element_type=jnp.float32,
        )
        o_ref[...] = acc * (sx_ref[0] * sw_ref[0])

    grid = (N // BN, M // BM)
    return pl.pallas_call(
        body,
        grid=grid,
        in_specs=[
            pl.BlockSpec(memory_space=pltpu.MemorySpace.SMEM),
            pl.BlockSpec(memory_space=pltpu.MemorySpace.SMEM),
            pl.BlockSpec((BM, K), lambda n, m: (m, 0)),
            pl.BlockSpec((K, BN), lambda n, m: (0, n)),
        ],
        out_specs=pl.BlockSpec((BM, BN), lambda n, m: (m, n)),
        out_shape=jax.ShapeDtypeStruct((M, N), jnp.float32),
        compiler_params=pltpu.CompilerParams(
            dimension_semantics=("arbitrary", "arbitrary"),
        ),
    )(scale_x, scale_w, x, w)


def _ring_allreduce(p):
    M, N = p.shape
    MC = M // N_DEV
    NQ = 2048
    NQUARTERS = N // NQ

    def body(p_ref, o_ref, acc, load, comm, send_sems, recv_sems,
             load_sem, store_sem):
        my = lax.axis_index("i")
        left = lax.rem(my + N_DEV - 1, N_DEV)
        right = lax.rem(my + 1, N_DEV)

        barrier_sem = pltpu.get_barrier_semaphore()
        for nbr in (left, right):
            pl.semaphore_signal(
                barrier_sem, inc=1,
                device_id=(nbr,), device_id_type=pl.DeviceIdType.MESH,
            )
        pl.semaphore_wait(barrier_sem, 2)

        g = 0
        for q in range(NQUARTERS):
            cols = pl.ds(q * NQ, NQ)

            cp = pltpu.make_async_copy(
                p_ref.at[pl.ds(my * MC, MC), cols], acc, load_sem)
            cp.start()
            cp.wait()

            for s in range(N_DEV - 1):
                slot = g % 2
                rdma = pltpu.make_async_remote_copy(
                    src_ref=acc,
                    dst_ref=comm.at[slot],
                    send_sem=send_sems.at[slot],
                    recv_sem=recv_sems.at[slot],
                    device_id=(right,),
                    device_id_type=pl.DeviceIdType.MESH,
                )
                rdma.start()
                recv_c = lax.rem(my - s - 1 + 2 * N_DEV, N_DEV)
                cp = pltpu.make_async_copy(
                    p_ref.at[pl.ds(recv_c * MC, MC), cols], load, load_sem)
                cp.start()
                rdma.wait()
                cp.wait()
                acc[...] = load[...] + comm[slot]
                g += 1

            rc = lax.rem(my + 1, N_DEV)
            st = pltpu.make_async_copy(
                acc, o_ref.at[pl.ds(rc * MC, MC), cols], store_sem)
            st.start()
            st.wait()

            src_buf = acc
            for s in range(N_DEV - 1):
                slot = g % 2
                rdma = pltpu.make_async_remote_copy(
                    src_ref=src_buf,
                    dst_ref=comm.at[slot],
                    send_sem=send_sems.at[slot],
                    recv_sem=recv_sems.at[slot],
                    device_id=(right,),
                    device_id_type=pl.DeviceIdType.MESH,
                )
                rdma.start()
                rdma.wait()
                gc = lax.rem(my - s + 2 * N_DEV, N_DEV)
                st = pltpu.make_async_copy(
                    comm.at[slot], o_ref.at[pl.ds(gc * MC, MC), cols],
                    store_sem)
                st.start()
                st.wait()
                src_buf = comm.at[slot]
                g += 1

    return pl.pallas_call(
        body,
        in_specs=[pl.BlockSpec(memory_space=pltpu.MemorySpace.HBM)],
        out_specs=pl.BlockSpec(memory_space=pltpu.MemorySpace.HBM),
        out_shape=jax.ShapeDtypeStruct((M, N), jnp.float32),
        scratch_shapes=[
            pltpu.VMEM((MC, NQ), jnp.float32),
            pltpu.VMEM((MC, NQ), jnp.float32),
            pltpu.VMEM((2, MC, NQ), jnp.float32),
            pltpu.SemaphoreType.DMA((2,)),
            pltpu.SemaphoreType.DMA((2,)),
            pltpu.SemaphoreType.DMA,
            pltpu.SemaphoreType.DMA,
        ],
        compiler_params=pltpu.CompilerParams(
            collective_id=0,
            vmem_limit_bytes=100 * 1024 * 1024,
        ),
    )(p)


def kernel(x, w_mat, scale_x, scale_w):
    p = _partial_matmul(x, w_mat, scale_x, scale_w)
    return _ring_allreduce(p)


# baseline (device time: 739299 ns/iter reference)
import jax
import jax.numpy as jnp
from jax import lax
from jax.experimental import pallas as pl
from jax.experimental.pallas import tpu as pltpu

N_DEV = 4


def _partial_matmul(x, w, scale_x, scale_w):
    M, K = x.shape
    _, N = w.shape
    BM, BN = 512, 2048

    def body(sx_ref, sw_ref, x_ref, w_ref, o_ref):
        a = x_ref[...].astype(jnp.bfloat16)
        b = w_ref[...].astype(jnp.bfloat16)
        acc = jax.lax.dot_general(
            a, b, (((1,), (0,)), ((), ())),
            preferred_element_type=jnp.float32,
        )
        o_ref[...] = (acc * (sx_ref[0] * sw_ref[0])).astype(jnp.bfloat16)

    grid = (N // BN, M // BM)
    return pl.pallas_call(
        body,
        grid=grid,
        in_specs=[
            pl.BlockSpec(memory_space=pltpu.MemorySpace.SMEM),
            pl.BlockSpec(memory_space=pltpu.MemorySpace.SMEM),
            pl.BlockSpec((BM, K), lambda n, m: (m, 0)),
            pl.BlockSpec((K, BN), lambda n, m: (0, n)),
        ],
        out_specs=pl.BlockSpec((BM, BN), lambda n, m: (m, n)),
        out_shape=jax.ShapeDtypeStruct((M, N), jnp.bfloat16),
        compiler_params=pltpu.CompilerParams(
            dimension_semantics=("arbitrary", "arbitrary"),
        ),
    )(scale_x, scale_w, x, w)


def _ring_allreduce(p):
    M, N = p.shape
    MC = M // N_DEV
    NQ = 2048
    HALF_QS = (N // 2) // NQ

    def body(p_ref, o_ref,
             acc_r, load_r, comm_r, acc_l, load_l, comm_l,
             send_sems_r, recv_sems_r, send_sems_l, recv_sems_l,
             load_sem_r, load_sem_l, store_sem_r, store_sem_l):
        my = lax.axis_index("i")
        left = lax.rem(my + N_DEV - 1, N_DEV)
        right = lax.rem(my + 1, N_DEV)

        barrier_sem = pltpu.get_barrier_semaphore()
        for nbr in (left, right):
            pl.semaphore_signal(
                barrier_sem, inc=1,
                device_id=(nbr,), device_id_type=pl.DeviceIdType.MESH,
            )
        pl.semaphore_wait(barrier_sem, 2)

        def row(c):
            return pl.ds(c * MC, MC)

        g = 0
        for q in range(HALF_QS):
            cols_r = pl.ds(q * NQ, NQ)
            cols_l = pl.ds((HALF_QS + q) * NQ, NQ)

            cp_r = pltpu.make_async_copy(
                p_ref.at[row(my), cols_r], acc_r, load_sem_r)
            cp_l = pltpu.make_async_copy(
                p_ref.at[row(my), cols_l], acc_l, load_sem_l)
            cp_r.start()
            cp_l.start()
            cp_r.wait()
            cp_l.wait()

            for s in range(N_DEV - 1):
                slot = g % 2
                rdma_r = pltpu.make_async_remote_copy(
                    src_ref=acc_r, dst_ref=comm_r.at[slot],
                    send_sem=send_sems_r.at[slot],
                    recv_sem=recv_sems_r.at[slot],
                    device_id=(right,), device_id_type=pl.DeviceIdType.MESH,
                )
                rdma_l = pltpu.make_async_remote_copy(
                    src_ref=acc_l, dst_ref=comm_l.at[slot],
                    send_sem=send_sems_l.at[slot],
                    recv_sem=recv_sems_l.at[slot],
                    device_id=(left,), device_id_type=pl.DeviceIdType.MESH,
                )
                rdma_r.start()
                rdma_l.start()
                recv_cr = lax.rem(my - s - 1 + 2 * N_DEV, N_DEV)
                recv_cl = lax.rem(my + s + 1, N_DEV)
                cp_r = pltpu.make_async_copy(
                    p_ref.at[row(recv_cr), cols_r], load_r, load_sem_r)
                cp_l = pltpu.make_async_copy(
                    p_ref.at[row(recv_cl), cols_l], load_l, load_sem_l)
                cp_r.start()
                cp_l.start()
                rdma_r.wait()
                rdma_l.wait()
                cp_r.wait()
                cp_l.wait()
                acc_r[...] = load_r[...] + comm_r[slot]
                acc_l[...] = load_l[...] + comm_l[slot]
                g += 1

            rc_r = lax.rem(my + 1, N_DEV)
            rc_l = lax.rem(my + N_DEV - 1, N_DEV)
            st_r = pltpu.make_async_copy(
                acc_r, o_ref.at[row(rc_r), cols_r], store_sem_r)
            st_l = pltpu.make_async_copy(
                acc_l, o_ref.at[row(rc_l), cols_l], store_sem_l)
            st_r.start()
            st_l.start()
            st_r.wait()
            st_l.wait()

            src_r, src_l = acc_r, acc_l
            for s in range(N_DEV - 1):
                slot = g % 2
                rdma_r = pltpu.make_async_remote_copy(
                    src_ref=src_r, dst_ref=comm_r.at[slot],
                    send_sem=send_sems_r.at[slot],
                    recv_sem=recv_sems_r.at[slot],
                    device_id=(right,), device_id_type=pl.DeviceIdType.MESH,
                )
                rdma_l = pltpu.make_async_remote_copy(
                    src_ref=src_l, dst_ref=comm_l.at[slot],
                    send_sem=send_sems_l.at[slot],
                    recv_sem=recv_sems_l.at[slot],
                    device_id=(left,), device_id_type=pl.DeviceIdType.MESH,
                )
                rdma_r.start()
                rdma_l.start()
                rdma_r.wait()
                rdma_l.wait()
                gc_r = lax.rem(my - s + 2 * N_DEV, N_DEV)
                gc_l = lax.rem(my + s, N_DEV)
                st_r = pltpu.make_async_copy(
                    comm_r.at[slot], o_ref.at[row(gc_r), cols_r], store_sem_r)
                st_l = pltpu.make_async_copy(
                    comm_l.at[slot], o_ref.at[row(gc_l), cols_l], store_sem_l)
                st_r.start()
                st_l.start()
                st_r.wait()
                st_l.wait()
                src_r = comm_r.at[slot]
                src_l = comm_l.at[slot]
                g += 1

    return pl.pallas_call(
        body,
        in_specs=[pl.BlockSpec(memory_space=pltpu.MemorySpace.HBM)],
        out_specs=pl.BlockSpec(memory_space=pltpu.MemorySpace.HBM),
        out_shape=jax.ShapeDtypeStruct((M, N), jnp.bfloat16),
        scratch_shapes=[
            pltpu.VMEM((MC, NQ), jnp.bfloat16),
            pltpu.VMEM((MC, NQ), jnp.bfloat16),
            pltpu.VMEM((2, MC, NQ), jnp.bfloat16),
            pltpu.VMEM((MC, NQ), jnp.bfloat16),
            pltpu.VMEM((MC, NQ), jnp.bfloat16),
            pltpu.VMEM((2, MC, NQ), jnp.bfloat16),
            pltpu.SemaphoreType.DMA((2,)),
            pltpu.SemaphoreType.DMA((2,)),
            pltpu.SemaphoreType.DMA((2,)),
            pltpu.SemaphoreType.DMA((2,)),
            pltpu.SemaphoreType.DMA,
            pltpu.SemaphoreType.DMA,
            pltpu.SemaphoreType.DMA,
            pltpu.SemaphoreType.DMA,
        ],
        compiler_params=pltpu.CompilerParams(
            collective_id=0,
            vmem_limit_bytes=100 * 1024 * 1024,
        ),
    )(p)


def kernel(x, w_mat, scale_x, scale_w):
    p = _partial_matmul(x, w_mat, scale_x, scale_w)
    return _ring_allreduce(p)
